# baseline (speedup 1.0000x reference)
import jax
import jax.numpy as jnp
from jax import lax
from jax.experimental import pallas as pl
from jax.experimental.pallas import tpu as pltpu
from jax.experimental.pallas import tpu_sc as plsc

_N = 100000
_S = 64
_L = 16
_NC = 2
_NS = 16
_NW = _NC * _NS
_CHUNK = 3200
_H1 = 1664
_H2 = _CHUNK - _H1
_ALIGNED = 99968
_TAIL = _N - _ALIGNED
_LAST_BASE = _ALIGNED - _CHUNK


def _sc_body(x_hbm, idx_hbm, shifts_hbm, scales_hbm, xt_hbm, tp_hbm,
             out_hbm,
             idx_v, x_v, o_v, sh_v, sc_v, idx_t_v, x_t_v, t_v, tp_v, sem):
    wid = lax.axis_index("s") * _NC + lax.axis_index("c")
    base = jnp.minimum(wid * _CHUNK, _LAST_BASE)

    c1 = pltpu.async_copy(shifts_hbm, sh_v, sem)
    c2 = pltpu.async_copy(scales_hbm, sc_v, sem)
    c3 = pltpu.async_copy(idx_hbm.at[pl.ds(base, _H1)],
                          idx_v.at[pl.ds(0, _H1)], sem)
    c4 = pltpu.async_copy(x_hbm.at[0, pl.ds(base, _H1)],
                          x_v.at[pl.ds(0, _H1)], sem)
    c5 = pltpu.async_copy(idx_hbm.at[pl.ds(base + _H1, _H2)],
                          idx_v.at[pl.ds(_H1, _H2)], sem)
    c6 = pltpu.async_copy(x_hbm.at[0, pl.ds(base + _H1, _H2)],
                          x_v.at[pl.ds(_H1, _H2)], sem)
    c1.wait()
    c2.wait()
    c3.wait()
    c4.wait()

    @plsc.parallel_loop(0, _H1, step=_L, unroll=4)
    def _step_lo(o):
        iv = idx_v[pl.ds(o, _L)]
        xv = x_v[pl.ds(o, _L)]
        sh = plsc.load_gather(sh_v, [iv])
        sc = plsc.load_gather(sc_v, [iv])
        o_v[pl.ds(o, _L)] = sh + sc * xv

    c5.wait()
    c6.wait()

    @plsc.parallel_loop(_H1, _CHUNK, step=_L, unroll=4)
    def _step_hi(o):
        iv = idx_v[pl.ds(o, _L)]
        xv = x_v[pl.ds(o, _L)]
        sh = plsc.load_gather(sh_v, [iv])
        sc = plsc.load_gather(sc_v, [iv])
        o_v[pl.ds(o, _L)] = sh + sc * xv

    pltpu.sync_copy(o_v, out_hbm.at[0, pl.ds(base, _CHUNK)])

    @pl.when(wid == _NW - 1)
    def _tail():
        ct1 = pltpu.async_copy(idx_hbm.at[pl.ds(_ALIGNED, _TAIL)],
                               idx_t_v, sem)
        ct2 = pltpu.async_copy(xt_hbm.at[0], x_t_v, sem)
        ct3 = pltpu.async_copy(tp_hbm, tp_v, sem)
        ct1.wait()
        ct2.wait()
        ct3.wait()

        @plsc.parallel_loop(0, _TAIL, step=_L)
        def _step_t(o):
            iv = idx_t_v[pl.ds(o, _L)]
            xv = x_t_v[pl.ds(o, _L)]
            sh = plsc.load_gather(sh_v, [iv])
            sc = plsc.load_gather(sc_v, [iv])
            t_v[pl.ds(o, _L)] = sh + sc * xv

        pltpu.sync_copy(t_v, out_hbm.at[0].at[tp_v])


@jax.jit
def _sc_shift(x_row, idx, shifts, scales, x_tail, tail_pos):
    mesh = plsc.VectorSubcoreMesh(core_axis_name="c", subcore_axis_name="s")
    fn = pl.kernel(
        _sc_body,
        out_type=jax.ShapeDtypeStruct((1, _N), jnp.float32),
        mesh=mesh,
        scratch_types=[
            pltpu.VMEM((_CHUNK,), jnp.int32),
            pltpu.VMEM((_CHUNK,), jnp.float32),
            pltpu.VMEM((_CHUNK,), jnp.float32),
            pltpu.VMEM((_S,), jnp.float32),
            pltpu.VMEM((_S,), jnp.float32),
            pltpu.VMEM((_TAIL,), jnp.int32),
            pltpu.VMEM((_TAIL,), jnp.float32),
            pltpu.VMEM((_TAIL,), jnp.float32),
            pltpu.VMEM((_TAIL,), jnp.int32),
            pltpu.SemaphoreType.DMA,
        ],
        compiler_params=pltpu.CompilerParams(needs_layout_passes=False),
    )
    return fn(x_row, idx, shifts, scales, x_tail, tail_pos)


def kernel(x, species_idx, shifts, scales):
    x_row = x.reshape(1, _N)
    x_tail = lax.slice(x, (_ALIGNED, 0), (_N, 1)).reshape(1, _TAIL)
    tail_pos = jnp.arange(_ALIGNED, _N, dtype=jnp.int32)
    out = _sc_shift(x_row, species_idx.astype(jnp.int32),
                    shifts, scales, x_tail, tail_pos)
    return out.reshape(_N, 1)


# speedup vs baseline: 1.1147x; 1.1147x over previous
import jax
import jax.numpy as jnp
from jax import lax
from jax.experimental import pallas as pl
from jax.experimental.pallas import tpu as pltpu
from jax.experimental.pallas import tpu_sc as plsc

_N = 100000
_S = 64
_L = 16
_NC = 2
_NS = 16
_NW = _NC * _NS
_CHUNK = 3200
_H1 = 1664
_H2 = _CHUNK - _H1
_ALIGNED = 99968
_TAIL = _N - _ALIGNED
_LAST_BASE = _ALIGNED - _CHUNK


def _sc_body(x_hbm, idx_hbm, shifts_hbm, scales_hbm, xt_hbm,
             out_hbm, tail_hbm,
             idx_v, x_v, o_v, sh_v, sc_v, idx_t_v, x_t_v, t_v, sem):
    wid = lax.axis_index("s") * _NC + lax.axis_index("c")
    base = jnp.minimum(wid * _CHUNK, _LAST_BASE)

    c1 = pltpu.async_copy(shifts_hbm, sh_v, sem)
    c2 = pltpu.async_copy(scales_hbm, sc_v, sem)
    c3 = pltpu.async_copy(idx_hbm.at[pl.ds(base, _H1)],
                          idx_v.at[pl.ds(0, _H1)], sem)
    c4 = pltpu.async_copy(x_hbm.at[0, pl.ds(base, _H1)],
                          x_v.at[pl.ds(0, _H1)], sem)
    c5 = pltpu.async_copy(idx_hbm.at[pl.ds(base + _H1, _H2)],
                          idx_v.at[pl.ds(_H1, _H2)], sem)
    c6 = pltpu.async_copy(x_hbm.at[0, pl.ds(base + _H1, _H2)],
                          x_v.at[pl.ds(_H1, _H2)], sem)
    c1.wait()
    c2.wait()
    c3.wait()
    c4.wait()

    @plsc.parallel_loop(0, _H1, step=_L, unroll=8)
    def _step_lo(o):
        iv = idx_v[pl.ds(o, _L)]
        xv = x_v[pl.ds(o, _L)]
        sh = plsc.load_gather(sh_v, [iv])
        sc = plsc.load_gather(sc_v, [iv])
        o_v[pl.ds(o, _L)] = sh + sc * xv

    c5.wait()
    c6.wait()

    @plsc.parallel_loop(_H1, _CHUNK, step=_L, unroll=8)
    def _step_hi(o):
        iv = idx_v[pl.ds(o, _L)]
        xv = x_v[pl.ds(o, _L)]
        sh = plsc.load_gather(sh_v, [iv])
        sc = plsc.load_gather(sc_v, [iv])
        o_v[pl.ds(o, _L)] = sh + sc * xv

    pltpu.sync_copy(o_v, out_hbm.at[0, pl.ds(base, _CHUNK)])

    @pl.when(wid == _NW - 1)
    def _tail():
        ct1 = pltpu.async_copy(idx_hbm.at[pl.ds(_ALIGNED, _TAIL)],
                               idx_t_v, sem)
        ct2 = pltpu.async_copy(xt_hbm.at[0], x_t_v, sem)
        ct1.wait()
        ct2.wait()

        @plsc.parallel_loop(0, _TAIL, step=_L)
        def _step_t(o):
            iv = idx_t_v[pl.ds(o, _L)]
            xv = x_t_v[pl.ds(o, _L)]
            sh = plsc.load_gather(sh_v, [iv])
            sc = plsc.load_gather(sc_v, [iv])
            t_v[pl.ds(o, _L)] = sh + sc * xv

        pltpu.sync_copy(t_v, tail_hbm.at[0])


@jax.jit
def _sc_shift(x_row, idx, shifts, scales, x_tail):
    mesh = plsc.VectorSubcoreMesh(core_axis_name="c", subcore_axis_name="s")
    fn = pl.kernel(
        _sc_body,
        out_type=(jax.ShapeDtypeStruct((1, _N), jnp.float32),
                  jax.ShapeDtypeStruct((1, _TAIL), jnp.float32)),
        mesh=mesh,
        scratch_types=[
            pltpu.VMEM((_CHUNK,), jnp.int32),
            pltpu.VMEM((_CHUNK,), jnp.float32),
            pltpu.VMEM((_CHUNK,), jnp.float32),
            pltpu.VMEM((_S,), jnp.float32),
            pltpu.VMEM((_S,), jnp.float32),
            pltpu.VMEM((_TAIL,), jnp.int32),
            pltpu.VMEM((_TAIL,), jnp.float32),
            pltpu.VMEM((_TAIL,), jnp.float32),
            pltpu.SemaphoreType.DMA,
        ],
        compiler_params=pltpu.CompilerParams(needs_layout_passes=False),
    )
    return fn(x_row, idx, shifts, scales, x_tail)


def kernel(x, species_idx, shifts, scales):
    x_row = x.reshape(1, _N)
    x_tail = lax.slice(x, (_ALIGNED, 0), (_N, 1)).reshape(1, _TAIL)
    out, tail = _sc_shift(x_row, species_idx.astype(jnp.int32),
                          shifts, scales, x_tail)
    out = lax.dynamic_update_slice(out, tail, (0, _ALIGNED))
    return out.reshape(_N, 1)


# speedup vs baseline: 1.1309x; 1.0145x over previous
import jax
import jax.numpy as jnp
from jax import lax
from jax.experimental import pallas as pl
from jax.experimental.pallas import tpu as pltpu
from jax.experimental.pallas import tpu_sc as plsc

_N = 100000
_S = 64
_L = 16
_NC = 2
_NS = 16
_NW = _NC * _NS
_CHUNK = 3200
_H1 = 1664
_H2 = _CHUNK - _H1
_ALIGNED = 99968
_TAIL = _N - _ALIGNED
_LAST_BASE = _ALIGNED - _CHUNK


def _sc_body(x_hbm, idx_hbm, shifts_hbm, scales_hbm, xt_hbm,
             out_hbm, tail_hbm,
             idx_v, x_v, o_v, sh_v, sc_v, idx_t_v, x_t_v, t_v, sem):
    wid = lax.axis_index("s") * _NC + lax.axis_index("c")
    base = jnp.minimum(wid * _CHUNK, _LAST_BASE)

    c1 = pltpu.async_copy(shifts_hbm, sh_v, sem)
    c2 = pltpu.async_copy(scales_hbm, sc_v, sem)
    c3 = pltpu.async_copy(idx_hbm.at[pl.ds(base, _H1)],
                          idx_v.at[pl.ds(0, _H1)], sem)
    c4 = pltpu.async_copy(x_hbm.at[0, pl.ds(base, _H1)],
                          x_v.at[pl.ds(0, _H1)], sem)
    c5 = pltpu.async_copy(idx_hbm.at[pl.ds(base + _H1, _H2)],
                          idx_v.at[pl.ds(_H1, _H2)], sem)
    c6 = pltpu.async_copy(x_hbm.at[0, pl.ds(base + _H1, _H2)],
                          x_v.at[pl.ds(_H1, _H2)], sem)
    c1.wait()
    c2.wait()
    c3.wait()
    c4.wait()

    @plsc.parallel_loop(0, _H1, step=_L, unroll=4)
    def _step_lo(o):
        iv = idx_v[pl.ds(o, _L)]
        xv = x_v[pl.ds(o, _L)]
        sh = plsc.load_gather(sh_v, [iv])
        sc = plsc.load_gather(sc_v, [iv])
        o_v[pl.ds(o, _L)] = sh + sc * xv

    c5.wait()
    c6.wait()

    @plsc.parallel_loop(_H1, _CHUNK, step=_L, unroll=4)
    def _step_hi(o):
        iv = idx_v[pl.ds(o, _L)]
        xv = x_v[pl.ds(o, _L)]
        sh = plsc.load_gather(sh_v, [iv])
        sc = plsc.load_gather(sc_v, [iv])
        o_v[pl.ds(o, _L)] = sh + sc * xv

    pltpu.sync_copy(o_v, out_hbm.at[0, pl.ds(base, _CHUNK)])

    @pl.when(wid == _NW - 1)
    def _tail():
        ct1 = pltpu.async_copy(idx_hbm.at[pl.ds(_ALIGNED, _TAIL)],
                               idx_t_v, sem)
        ct2 = pltpu.async_copy(xt_hbm.at[0], x_t_v, sem)
        ct1.wait()
        ct2.wait()

        @plsc.parallel_loop(0, _TAIL, step=_L)
        def _step_t(o):
            iv = idx_t_v[pl.ds(o, _L)]
            xv = x_t_v[pl.ds(o, _L)]
            sh = plsc.load_gather(sh_v, [iv])
            sc = plsc.load_gather(sc_v, [iv])
            t_v[pl.ds(o, _L)] = sh + sc * xv

        pltpu.sync_copy(t_v, tail_hbm.at[0])


@jax.jit
def _sc_shift(x_row, idx, shifts, scales, x_tail):
    mesh = plsc.VectorSubcoreMesh(core_axis_name="c", subcore_axis_name="s")
    fn = pl.kernel(
        _sc_body,
        out_type=(jax.ShapeDtypeStruct((1, _N), jnp.float32),
                  jax.ShapeDtypeStruct((1, _TAIL), jnp.float32)),
        mesh=mesh,
        scratch_types=[
            pltpu.VMEM((_CHUNK,), jnp.int32),
            pltpu.VMEM((_CHUNK,), jnp.float32),
            pltpu.VMEM((_CHUNK,), jnp.float32),
            pltpu.VMEM((_S,), jnp.float32),
            pltpu.VMEM((_S,), jnp.float32),
            pltpu.VMEM((_TAIL,), jnp.int32),
            pltpu.VMEM((_TAIL,), jnp.float32),
            pltpu.VMEM((_TAIL,), jnp.float32),
            pltpu.SemaphoreType.DMA,
        ],
        compiler_params=pltpu.CompilerParams(needs_layout_passes=False),
    )
    return fn(x_row, idx, shifts, scales, x_tail)


def kernel(x, species_idx, shifts, scales):
    x_row = x.reshape(1, _N)
    x_tail = lax.slice(x, (_ALIGNED, 0), (_N, 1)).reshape(1, _TAIL)
    out, tail = _sc_shift(x_row, species_idx.astype(jnp.int32),
                          shifts, scales, x_tail)
    out = lax.dynamic_update_slice(out, tail, (0, _ALIGNED))
    return out.reshape(_N, 1)
